# Initial kernel scaffold; baseline (speedup 1.0000x reference)
import jax
import jax.numpy as jnp
from jax.experimental import pallas as pl


def kernel(next_positions, position_sequence_noise, position_sequence, nparticles_per_example, particle_types, receivers, senders, params):
    raise NotImplementedError("write your pallas kernel here")


# speedup vs baseline: 2.1503x; 2.1503x over previous
import functools

import jax
import jax.numpy as jnp
from jax import lax
from jax.experimental import pallas as pl
from jax.experimental.pallas import tpu as pltpu
from jax.experimental.pallas import tpu_sc as plsc

RADIUS = 0.06
LN_EPS = 1e-5
LAT = 128
BLK = 2048


def _rsqrt_exact(v):
    r0 = lax.rsqrt(v)
    return r0 * (1.5 - 0.5 * v * r0 * r0)


def _ln(v, g, b):
    mu = jnp.mean(v, axis=-1, keepdims=True)
    var = jnp.mean((v - mu) ** 2, axis=-1, keepdims=True)
    return (v - mu) * _rsqrt_exact(var + LN_EPS) * g + b


def _dot(a, b):
    return jnp.dot(a, b, preferred_element_type=jnp.float32,
                   precision=lax.Precision.HIGHEST)


def _dotd(a, b):
    return jnp.dot(a, b, preferred_element_type=jnp.float32)


def _sc_gather2(tab_a, tab_b, idx_a, idx_b, d, epad):
    nblk = epad // 128 // 32
    chunk = nblk * 128
    mesh = plsc.VectorSubcoreMesh(core_axis_name="c", subcore_axis_name="s")

    @functools.partial(
        pl.kernel,
        out_type=[jax.ShapeDtypeStruct((epad, d), jnp.float32)] * 2,
        mesh=mesh,
        scratch_types=[
            pltpu.VMEM((128,), jnp.int32),
            pltpu.VMEM((128,), jnp.int32),
            pltpu.VMEM((128, d), jnp.float32),
            pltpu.VMEM((128, d), jnp.float32),
        ],
    )
    def k(ta, tb, ia, ib, oa, ob, ia_v, ib_v, buf_a, buf_b):
        c = lax.axis_index("c")
        s = lax.axis_index("s")
        wid = s * 2 + c
        base = wid * chunk

        def body(j, _):
            rows = pl.ds(base + j * 128, 128)
            pltpu.sync_copy(ia.at[rows], ia_v)
            pltpu.sync_copy(ib.at[rows], ib_v)
            pltpu.sync_copy(ta.at[ia_v], buf_a)
            pltpu.sync_copy(tb.at[ib_v], buf_b)
            pltpu.sync_copy(buf_a, oa.at[rows])
            pltpu.sync_copy(buf_b, ob.at[rows])
            return 0

        lax.fori_loop(0, nblk, body, 0)

    return k(tab_a, tab_b, idx_a, idx_b)


def _sc_scatter(e_new, idx_r, zeros_init, epad, n):
    half = n // 2
    trash = 128
    h = half + trash
    rows_init = h // 16
    rows_out = half // 16
    nblk = epad // 128 // 16
    chunk = nblk * 128
    mesh = plsc.VectorSubcoreMesh(core_axis_name="c", subcore_axis_name="s")

    @functools.partial(
        pl.kernel,
        out_type=jax.ShapeDtypeStruct((n, LAT), jnp.float32),
        mesh=mesh,
        scratch_types=[
            pltpu.VMEM((chunk,), jnp.int32),
            pltpu.VMEM((128,), jnp.int32),
            pltpu.VMEM((128, LAT), jnp.float32),
            pltpu.VMEM_SHARED((h, LAT), jnp.float32),
        ],
    )
    def k(enew_h, ridx_h, zinit_h, out, ridx_v, loc_v, buf, acc):
        c = lax.axis_index("c")
        s = lax.axis_index("s")
        base = c * half
        pltpu.sync_copy(zinit_h, acc.at[pl.ds(s * rows_init, rows_init)])
        pltpu.sync_copy(ridx_h.at[pl.ds(s * chunk, chunk)], ridx_v)
        plsc.subcore_barrier()

        def overlaps(j):
            first = ridx_v[pl.ds(j * 128, 16)][0]
            last = ridx_v[pl.ds(j * 128 + 112, 16)][15]
            return (first < base + half) & (last >= base)

        def body(j, _):
            @pl.when(overlaps(j))
            def _():
                pltpu.sync_copy(
                    enew_h.at[pl.ds((s * nblk + j) * 128, 128)], buf)
                for q in range(8):
                    sl = pl.ds(q * 16, 16)
                    v = ridx_v[pl.ds(j * 128 + q * 16, 16)]
                    inr = (v >= base) & (v < base + half)
                    loc_v[sl] = jnp.where(inr, v - base, half + s)
                pltpu.sync_copy(buf, acc.at[loc_v], add=True)
            return 0

        lax.fori_loop(0, nblk, body, 0)
        plsc.subcore_barrier()
        pltpu.sync_copy(
            acc.at[pl.ds(s * rows_out, rows_out)],
            out.at[pl.ds(base + s * rows_out, rows_out)],
        )

    return k(e_new, idx_r, zeros_init)


def _row_spec():
    return pl.BlockSpec((BLK, LAT), lambda i: (i, 0))


def _full(shape):
    return pl.BlockSpec(shape, lambda i: tuple(0 for _ in shape))


def _node_encoder(nf16, ptypes, temb, w1a, w1e, b1, w2, b2, w3, b3, g, bl,
                  wes, wer, n):
    def body(nf_ref, pt_ref, te_ref, w1a_ref, w1e_ref, b1_ref, w2_ref, b2_ref,
             w3_ref, b3_ref, g_ref, bl_ref, wes_ref, wer_ref,
             x_ref, ps_ref, pr_ref):
        t9 = _dot(te_ref[...], w1e_ref[...])
        iot = lax.broadcasted_iota(jnp.int32, (BLK, 9), 1)
        oh = (pt_ref[...] == iot).astype(jnp.float32)
        hh = _dot(nf_ref[...], w1a_ref[...]) + _dot(oh, t9) + b1_ref[...]
        hh = jnp.maximum(hh, 0.0)
        hh = jnp.maximum(_dot(hh, w2_ref[...]) + b2_ref[...], 0.0)
        v = _ln(_dot(hh, w3_ref[...]) + b3_ref[...], g_ref[...], bl_ref[...])
        x_ref[...] = v
        ps_ref[...] = _dotd(v, wes_ref[...])
        pr_ref[...] = _dotd(v, wer_ref[...])

    return pl.pallas_call(
        body,
        grid=(n // BLK,),
        in_specs=[
            pl.BlockSpec((BLK, 16), lambda i: (i, 0)),
            pl.BlockSpec((BLK, 1), lambda i: (i, 0)),
            _full((9, 16)), _full((16, LAT)), _full((16, LAT)),
            _full((1, LAT)), _full((LAT, LAT)), _full((1, LAT)),
            _full((LAT, LAT)), _full((1, LAT)), _full((1, LAT)),
            _full((1, LAT)), _full((LAT, LAT)), _full((LAT, LAT)),
        ],
        out_specs=[_row_spec(), _row_spec(), _row_spec()],
        out_shape=[jax.ShapeDtypeStruct((n, LAT), jnp.float32)] * 3,
    )(nf16, ptypes, temb, w1a, w1e, b1, w2, b2, w3, b3, g, bl, wes, wer)


def _edge_encoder(rel_a, rel_b, w1p, w1d, b1, w2, b2, w3, b3, g, bl, epad):
    def body(ra_ref, rb_ref, w1p_ref, w1d_ref, b1_ref, w2_ref, b2_ref,
             w3_ref, b3_ref, g_ref, bl_ref, e_ref):
        r = ra_ref[...] + rb_ref[...]
        d2 = jnp.sum(r * r, axis=-1, keepdims=True)
        dist = jnp.where(d2 > 1e-24, d2 * _rsqrt_exact(d2), 0.0)
        hh = _dotd(r, w1p_ref[...]) + dist * w1d_ref[...] + b1_ref[...]
        hh = jnp.maximum(hh, 0.0)
        hh = jnp.maximum(_dotd(hh, w2_ref[...]) + b2_ref[...], 0.0)
        e_ref[...] = _ln(_dotd(hh, w3_ref[...]) + b3_ref[...], g_ref[...],
                         bl_ref[...])

    return pl.pallas_call(
        body,
        grid=(epad // BLK,),
        in_specs=[
            pl.BlockSpec((BLK, LAT), lambda i: (i, 0)),
            pl.BlockSpec((BLK, LAT), lambda i: (i, 0)),
            _full((LAT, LAT)), _full((1, LAT)), _full((1, LAT)),
            _full((LAT, LAT)), _full((1, LAT)),
            _full((LAT, LAT)), _full((1, LAT)),
            _full((1, LAT)), _full((1, LAT)),
        ],
        out_specs=[_row_spec()],
        out_shape=[jax.ShapeDtypeStruct((epad, LAT), jnp.float32)],
    )(rel_a, rel_b, w1p, w1d, b1, w2, b2, w3, b3, g, bl)[0]


def _edge_step(e, gs_a, gs_b, we, b1, w2, b2, w3, b3, g, bl, n_edges, epad,
               last):
    def body(e_ref, gsa_ref, gsb_ref, we_ref, b1_ref, w2_ref, b2_ref,
             w3_ref, b3_ref, g_ref, bl_ref, *o_refs):
        eb = e_ref[...]
        hh = (_dotd(eb, we_ref[...]) + gsa_ref[...] + gsb_ref[...]
              + b1_ref[...])
        hh = jnp.maximum(hh, 0.0)
        hh = jnp.maximum(_dotd(hh, w2_ref[...]) + b2_ref[...], 0.0)
        v = _ln(_dotd(hh, w3_ref[...]) + b3_ref[...], g_ref[...], bl_ref[...])
        row = (pl.program_id(0) * BLK
               + lax.broadcasted_iota(jnp.int32, (BLK, LAT), 0))
        v = jnp.where(row < n_edges, v, 0.0)
        o_refs[0][...] = v
        if not last:
            o_refs[1][...] = eb + v

    n_out = 1 if last else 2
    return pl.pallas_call(
        body,
        grid=(epad // BLK,),
        in_specs=[
            _row_spec(), _row_spec(), _row_spec(),
            _full((LAT, LAT)), _full((1, LAT)),
            _full((LAT, LAT)), _full((1, LAT)),
            _full((LAT, LAT)), _full((1, LAT)),
            _full((1, LAT)), _full((1, LAT)),
        ],
        out_specs=[_row_spec()] * n_out,
        out_shape=[jax.ShapeDtypeStruct((epad, LAT), jnp.float32)] * n_out,
    )(e, gs_a, gs_b, we, b1, w2, b2, w3, b3, g, bl)


def _node_step(x, agg, w1x, w1a, b1, w2, b2, w3, b3, g, bl, wes, wer, n, last):
    def body(x_ref, a_ref, w1x_ref, w1a_ref, b1_ref, w2_ref, b2_ref, w3_ref,
             b3_ref, g_ref, bl_ref, wes_ref, wer_ref, *o_refs):
        xb = x_ref[...]
        hh = (_dotd(xb, w1x_ref[...]) + _dotd(a_ref[...], w1a_ref[...])
              + b1_ref[...])
        hh = jnp.maximum(hh, 0.0)
        hh = jnp.maximum(_dotd(hh, w2_ref[...]) + b2_ref[...], 0.0)
        v = _ln(_dotd(hh, w3_ref[...]) + b3_ref[...], g_ref[...], bl_ref[...])
        xn = xb + v
        o_refs[0][...] = xn
        if not last:
            o_refs[1][...] = _dotd(xn, wes_ref[...])
            o_refs[2][...] = _dotd(xn, wer_ref[...])

    n_out = 1 if last else 3
    return pl.pallas_call(
        body,
        grid=(n // BLK,),
        in_specs=[
            _row_spec(), _row_spec(),
            _full((LAT, LAT)), _full((LAT, LAT)), _full((1, LAT)),
            _full((LAT, LAT)), _full((1, LAT)),
            _full((LAT, LAT)), _full((1, LAT)),
            _full((1, LAT)), _full((1, LAT)),
            _full((LAT, LAT)), _full((LAT, LAT)),
        ],
        out_specs=[_row_spec()] * n_out,
        out_shape=[jax.ShapeDtypeStruct((n, LAT), jnp.float32)] * n_out,
    )(x, agg, w1x, w1a, b1, w2, b2, w3, b3, g, bl, wes, wer)


def _decoder(x, w1, b1, w2, b2, wd, bd, nxt, nl, yl, yp, n):
    def body(x_ref, w1_ref, b1_ref, w2_ref, b2_ref, wd_ref, bd_ref,
             nx_ref, nl_ref, yl_ref, yp_ref, p_ref, t_ref):
        hh = jnp.maximum(_dotd(x_ref[...], w1_ref[...]) + b1_ref[...], 0.0)
        hh = jnp.maximum(_dotd(hh, w2_ref[...]) + b2_ref[...], 0.0)
        p_ref[...] = _dotd(hh, wd_ref[...]) + bd_ref[...]
        t_ref[...] = (nx_ref[...] + nl_ref[...]
                      - 2.0 * yl_ref[...] + yp_ref[...])

    two = pl.BlockSpec((BLK, 2), lambda i: (i, 0))
    return pl.pallas_call(
        body,
        grid=(n // BLK,),
        in_specs=[
            _row_spec(),
            _full((LAT, LAT)), _full((1, LAT)),
            _full((LAT, LAT)), _full((1, LAT)),
            _full((LAT, 2)), _full((1, 2)),
            two, two, two, two,
        ],
        out_specs=[two, two],
        out_shape=[jax.ShapeDtypeStruct((n, 2), jnp.float32)] * 2,
    )(x, w1, b1, w2, b2, wd, bd, nxt, nl, yl, yp)


def _r1(b):
    return b.reshape(1, -1)


def kernel(next_positions, position_sequence_noise, position_sequence,
           nparticles_per_example, particle_types, receivers, senders, params):
    n = position_sequence.shape[0]
    e_cnt = receivers.shape[0]
    epad = max(8192, -(-e_cnt // 8192) * 8192)
    pad = epad - e_cnt

    noisy = position_sequence + position_sequence_noise
    pos = noisy[:, -1]
    vel = (noisy[:, 1:] - noisy[:, :-1]).reshape(n, 10)
    dl = pos / RADIUS
    du = (1.0 - pos) / RADIUS
    bound = jnp.clip(jnp.concatenate([dl, du], axis=-1), -1.0, 1.0)
    nf16 = jnp.concatenate(
        [vel, bound, jnp.zeros((n, 2), jnp.float32)], axis=-1)

    s2d = jnp.concatenate(
        [senders, jnp.full((pad,), n - 1, senders.dtype)]).astype(jnp.int32)
    r2d = jnp.concatenate(
        [receivers, jnp.full((pad,), n - 1, receivers.dtype)]).astype(jnp.int32)

    posp = jnp.zeros((n, LAT), jnp.float32).at[:, 0:2].set(pos / RADIUS)
    rel_a, rel_b = _sc_gather2(posp, -posp, s2d, r2d, LAT, epad)

    p = params
    ee = p["edge_enc"]["layers"]
    w1 = ee[0][0]
    w1p = jnp.zeros((LAT, LAT), jnp.float32).at[0:2].set(w1[0:2])
    e = _edge_encoder(
        rel_a, rel_b, w1p, _r1(w1[2]), _r1(ee[0][1]), ee[1][0], _r1(ee[1][1]),
        ee[2][0], _r1(ee[2][1]), _r1(p["edge_enc"]["ln_g"]),
        _r1(p["edge_enc"]["ln_b"]), epad)

    ne = p["node_enc"]["layers"]
    w1n = ne[0][0]
    w1a = jnp.zeros((16, LAT), jnp.float32).at[0:14].set(w1n[0:14])
    w1e = w1n[14:30]
    we0 = p["gn"][0]["edge"]["layers"][0][0]
    x, ps, pr = _node_encoder(
        nf16, particle_types.reshape(n, 1).astype(jnp.int32), p["type_emb"],
        w1a, w1e, _r1(ne[0][1]), ne[1][0], _r1(ne[1][1]), ne[2][0],
        _r1(ne[2][1]), _r1(p["node_enc"]["ln_g"]), _r1(p["node_enc"]["ln_b"]),
        we0[128:256], we0[256:384], n)

    zeros_init = jnp.zeros(((n // 2 + 128) // 16, LAT), jnp.float32)
    n_steps = len(p["gn"])
    for t in range(n_steps):
        gp = p["gn"][t]
        el = gp["edge"]["layers"]
        nl_ = gp["node"]["layers"]
        last = t == n_steps - 1

        gs_a, gs_b = _sc_gather2(ps, pr, s2d, r2d, LAT, epad)
        eo = _edge_step(
            e, gs_a, gs_b, el[0][0][0:128], _r1(el[0][1]), el[1][0], _r1(el[1][1]),
            el[2][0], _r1(el[2][1]), _r1(gp["edge"]["ln_g"]),
            _r1(gp["edge"]["ln_b"]), e_cnt, epad, last)
        e_new = eo[0]
        if not last:
            e = eo[1]
        agg = _sc_scatter(e_new, r2d, zeros_init, epad, n)
        if not last:
            wen = p["gn"][t + 1]["edge"]["layers"][0][0]
            x, ps, pr = _node_step(
                x, agg, nl_[0][0][0:128], nl_[0][0][128:256], _r1(nl_[0][1]),
                nl_[1][0], _r1(nl_[1][1]), nl_[2][0], _r1(nl_[2][1]),
                _r1(gp["node"]["ln_g"]), _r1(gp["node"]["ln_b"]),
                wen[128:256], wen[256:384], n, last=False)
        else:
            (x,) = _node_step(
                x, agg, nl_[0][0][0:128], nl_[0][0][128:256], _r1(nl_[0][1]),
                nl_[1][0], _r1(nl_[1][1]), nl_[2][0], _r1(nl_[2][1]),
                _r1(gp["node"]["ln_g"]), _r1(gp["node"]["ln_b"]),
                jnp.zeros((LAT, LAT), jnp.float32),
                jnp.zeros((LAT, LAT), jnp.float32), n, last=True)

    dec = p["dec"]["layers"]
    pred_acc, target_acc = _decoder(
        x, dec[0][0], _r1(dec[0][1]), dec[1][0], _r1(dec[1][1]), dec[2][0],
        _r1(dec[2][1]), next_positions, position_sequence_noise[:, -1],
        noisy[:, -1], noisy[:, -2], n)
    return pred_acc, target_acc
